# Initial kernel scaffold; baseline (speedup 1.0000x reference)
import functools
import jax
import jax.numpy as jnp
from jax.experimental import pallas as pl

T = 2048
D = 1024
E = 8
F = 512
TOPK = 2
NG = 2
NSHARED = 2
SCALE = 2.5
FS = F * NSHARED
EHAT = E + NSHARED

BT = 512


def _router_body(x_ref, gw_ref, eb_ref, cmb_ref):
    x = x_ref[...]
    logits = jnp.dot(x, gw_ref[...])
    scores = jax.nn.sigmoid(logits)
    sfc = scores + eb_ref[...]

    grp = sfc.reshape(T * NG, E // NG)
    g1 = jnp.max(grp, axis=-1, keepdims=True)
    eq1 = (grp == g1).astype(jnp.float32)
    first1 = (eq1 * jnp.cumsum(eq1, axis=-1)) == 1.0
    g2 = jnp.max(jnp.where(first1, -jnp.inf, grp), axis=-1, keepdims=True)
    gs = (g1 + g2).reshape(T, NG)
    gsel = gs[:, 0:1] >= gs[:, 1:2]
    group_of_e = jax.lax.broadcasted_iota(jnp.int32, (T, E), 1) // (E // NG)
    emask = jnp.where(gsel, group_of_e == 0, group_of_e == 1)

    masked = jnp.where(emask, sfc, -1e9)
    m1 = jnp.max(masked, axis=-1, keepdims=True)
    e1 = (masked == m1).astype(jnp.float32)
    f1 = (e1 * jnp.cumsum(e1, axis=-1)) == 1.0
    masked2 = jnp.where(f1, -jnp.inf, masked)
    m2 = jnp.max(masked2, axis=-1, keepdims=True)
    e2 = (masked2 == m2).astype(jnp.float32)
    f2 = (e2 * jnp.cumsum(e2, axis=-1)) == 1.0

    w1 = jnp.sum(jnp.where(f1, scores, 0.0), axis=-1, keepdims=True)
    w2 = jnp.sum(jnp.where(f2, scores, 0.0), axis=-1, keepdims=True)
    denom = w1 + w2 + 1e-20
    cmb_ref[...] = (jnp.where(f1, w1, 0.0) + jnp.where(f2, w2, 0.0)) / denom


def _moe_body(x_ref, wgu_ref, wdn_ref, cmb_ref, o_ref):
    e = pl.program_id(1)

    @pl.when(e == 0)
    def _():
        o_ref[...] = jnp.zeros_like(o_ref)

    xb = x_ref[...]
    gu = jnp.dot(xb, wgu_ref[0], preferred_element_type=jnp.float32)
    g = gu[:, :F]
    h = (g * jax.nn.sigmoid(g) * gu[:, F:]).astype(jnp.bfloat16)
    y = jnp.dot(h, wdn_ref[0], preferred_element_type=jnp.float32)
    onehot = (jax.lax.broadcasted_iota(jnp.int32, (EHAT, 1), 0) == e).astype(
        jnp.float32)
    w = jnp.dot(cmb_ref[...], onehot)
    o_ref[...] += y * w


@jax.jit
def kernel(hidden_states, gate_w, e_bias, w_gate_up, w_down, ws_gate_up,
           ws_down):
    x = hidden_states.reshape(T, D)

    combine = pl.pallas_call(
        _router_body,
        out_shape=jax.ShapeDtypeStruct((T, E), jnp.float32),
    )(x, gate_w, e_bias.reshape(1, E))

    sg0 = ws_gate_up[:, 0:F]
    sg1 = ws_gate_up[:, F:FS]
    su0 = ws_gate_up[:, FS:FS + F]
    su1 = ws_gate_up[:, FS + F:]
    shared_gu = jnp.stack(
        [jnp.concatenate([sg0, su0], axis=1),
         jnp.concatenate([sg1, su1], axis=1)])
    wgu = jnp.concatenate([w_gate_up, shared_gu], axis=0).astype(jnp.bfloat16)
    shared_dn = jnp.stack([ws_down[:F], ws_down[F:]])
    wdn = jnp.concatenate([w_down, shared_dn], axis=0).astype(jnp.bfloat16)
    cmb = jnp.concatenate(
        [combine * SCALE, jnp.ones((T, NSHARED), jnp.float32)], axis=1)
    xb = x.astype(jnp.bfloat16)

    out = pl.pallas_call(
        _moe_body,
        grid=(T // BT, EHAT),
        in_specs=[
            pl.BlockSpec((BT, D), lambda i, e: (i, 0)),
            pl.BlockSpec((1, D, 2 * F), lambda i, e: (e, 0, 0)),
            pl.BlockSpec((1, F, D), lambda i, e: (e, 0, 0)),
            pl.BlockSpec((BT, EHAT), lambda i, e: (i, 0)),
        ],
        out_specs=pl.BlockSpec((BT, D), lambda i, e: (i, 0)),
        out_shape=jax.ShapeDtypeStruct((T, D), jnp.float32),
    )(xb, wgu, wdn, cmb)
    return out


# speedup vs baseline: 1.7512x; 1.7512x over previous
import functools
import jax
import jax.numpy as jnp
from jax.experimental import pallas as pl

T = 2048
D = 1024
E = 8
F = 512
TOPK = 2
NG = 2
NSHARED = 2
SCALE = 2.5
FS = F * NSHARED
EHAT = E + NSHARED

BT = 512


def _lane_cumsum(a):
    cols = [a[:, 0:1]]
    for j in range(1, a.shape[1]):
        cols.append(cols[-1] + a[:, j:j + 1])
    return jnp.concatenate(cols, axis=1)


def _router_body(x_ref, gw_ref, eb_ref, cmb_ref):
    x = x_ref[...]
    logits = jnp.dot(x, gw_ref[...])
    scores = jax.nn.sigmoid(logits)
    sfc = scores + eb_ref[...]

    def top2sum(grp):
        g1 = jnp.max(grp, axis=-1, keepdims=True)
        eq1 = jnp.where(grp == g1, 1.0, 0.0)
        first1 = (eq1 * _lane_cumsum(eq1)) == 1.0
        g2 = jnp.max(jnp.where(first1, -jnp.inf, grp), axis=-1, keepdims=True)
        return g1 + g2

    gs0 = top2sum(sfc[:, :E // NG])
    gs1 = top2sum(sfc[:, E // NG:])
    gsel_f = jnp.where(gs0 >= gs1, 1.0, 0.0)
    gof = (jax.lax.broadcasted_iota(jnp.int32, (T, E), 1) // (E // NG)
           ).astype(jnp.float32)
    emask_f = gsel_f * (1.0 - gof) + (1.0 - gsel_f) * gof

    masked = jnp.where(emask_f > 0.5, sfc, -1e9)
    m1 = jnp.max(masked, axis=-1, keepdims=True)
    e1 = jnp.where(masked == m1, 1.0, 0.0)
    f1 = (e1 * _lane_cumsum(e1)) == 1.0
    masked2 = jnp.where(f1, -jnp.inf, masked)
    m2 = jnp.max(masked2, axis=-1, keepdims=True)
    e2 = jnp.where(masked2 == m2, 1.0, 0.0)
    f2 = (e2 * _lane_cumsum(e2)) == 1.0

    w1 = jnp.sum(jnp.where(f1, scores, 0.0), axis=-1, keepdims=True)
    w2 = jnp.sum(jnp.where(f2, scores, 0.0), axis=-1, keepdims=True)
    denom = w1 + w2 + 1e-20
    cmb_ref[...] = (jnp.where(f1, w1, 0.0) + jnp.where(f2, w2, 0.0)) / denom


def _moe_body(x_ref, wgu_ref, wdn_ref, cmb_ref, o_ref):
    e = pl.program_id(1)

    @pl.when(e == 0)
    def _():
        o_ref[...] = jnp.zeros_like(o_ref)

    xb = x_ref[...]
    gu = jnp.dot(xb, wgu_ref[0], preferred_element_type=jnp.float32)
    g = gu[:, :F]
    h = (g * jax.nn.sigmoid(g) * gu[:, F:]).astype(jnp.bfloat16)
    y = jnp.dot(h, wdn_ref[0], preferred_element_type=jnp.float32)
    onehot = jnp.where(
        jax.lax.broadcasted_iota(jnp.int32, (EHAT, 1), 0) == e, 1.0, 0.0)
    w = jnp.dot(cmb_ref[...], onehot)
    o_ref[...] += y * w


@jax.jit
def kernel(hidden_states, gate_w, e_bias, w_gate_up, w_down, ws_gate_up,
           ws_down):
    x = hidden_states.reshape(T, D)

    combine = pl.pallas_call(
        _router_body,
        out_shape=jax.ShapeDtypeStruct((T, E), jnp.float32),
    )(x, gate_w, e_bias.reshape(1, E))

    sg0 = ws_gate_up[:, 0:F]
    sg1 = ws_gate_up[:, F:FS]
    su0 = ws_gate_up[:, FS:FS + F]
    su1 = ws_gate_up[:, FS + F:]
    shared_gu = jnp.stack(
        [jnp.concatenate([sg0, su0], axis=1),
         jnp.concatenate([sg1, su1], axis=1)])
    wgu = jnp.concatenate([w_gate_up, shared_gu], axis=0).astype(jnp.bfloat16)
    shared_dn = jnp.stack([ws_down[:F], ws_down[F:]])
    wdn = jnp.concatenate([w_down, shared_dn], axis=0).astype(jnp.bfloat16)
    cmb = jnp.concatenate(
        [combine * SCALE, jnp.ones((T, NSHARED), jnp.float32)], axis=1)
    xb = x.astype(jnp.bfloat16)

    out = pl.pallas_call(
        _moe_body,
        grid=(T // BT, EHAT),
        in_specs=[
            pl.BlockSpec((BT, D), lambda i, e: (i, 0)),
            pl.BlockSpec((1, D, 2 * F), lambda i, e: (e, 0, 0)),
            pl.BlockSpec((1, F, D), lambda i, e: (e, 0, 0)),
            pl.BlockSpec((BT, EHAT), lambda i, e: (i, 0)),
        ],
        out_specs=pl.BlockSpec((BT, D), lambda i, e: (i, 0)),
        out_shape=jax.ShapeDtypeStruct((T, D), jnp.float32),
    )(xb, wgu, wdn, cmb)
    return out


# speedup vs baseline: 1.7911x; 1.0228x over previous
import functools
import jax
import jax.numpy as jnp
from jax.experimental import pallas as pl

T = 2048
D = 1024
E = 8
F = 512
TOPK = 2
NG = 2
NSHARED = 2
SCALE = 2.5
FS = F * NSHARED
EHAT = E + NSHARED

BT = 2048


def _lane_cumsum(a):
    cols = [a[:, 0:1]]
    for j in range(1, a.shape[1]):
        cols.append(cols[-1] + a[:, j:j + 1])
    return jnp.concatenate(cols, axis=1)


def _router_body(x_ref, gw_ref, eb_ref, cmb_ref):
    x = x_ref[...]
    logits = jnp.dot(x, gw_ref[...])
    scores = jax.nn.sigmoid(logits)
    sfc = scores + eb_ref[...]

    def top2sum(grp):
        g1 = jnp.max(grp, axis=-1, keepdims=True)
        eq1 = jnp.where(grp == g1, 1.0, 0.0)
        first1 = (eq1 * _lane_cumsum(eq1)) == 1.0
        g2 = jnp.max(jnp.where(first1, -jnp.inf, grp), axis=-1, keepdims=True)
        return g1 + g2

    gs0 = top2sum(sfc[:, :E // NG])
    gs1 = top2sum(sfc[:, E // NG:])
    gsel_f = jnp.where(gs0 >= gs1, 1.0, 0.0)
    gof = (jax.lax.broadcasted_iota(jnp.int32, (T, E), 1) // (E // NG)
           ).astype(jnp.float32)
    emask_f = gsel_f * (1.0 - gof) + (1.0 - gsel_f) * gof

    masked = jnp.where(emask_f > 0.5, sfc, -1e9)
    m1 = jnp.max(masked, axis=-1, keepdims=True)
    e1 = jnp.where(masked == m1, 1.0, 0.0)
    f1 = (e1 * _lane_cumsum(e1)) == 1.0
    masked2 = jnp.where(f1, -jnp.inf, masked)
    m2 = jnp.max(masked2, axis=-1, keepdims=True)
    e2 = jnp.where(masked2 == m2, 1.0, 0.0)
    f2 = (e2 * _lane_cumsum(e2)) == 1.0

    w1 = jnp.sum(jnp.where(f1, scores, 0.0), axis=-1, keepdims=True)
    w2 = jnp.sum(jnp.where(f2, scores, 0.0), axis=-1, keepdims=True)
    denom = w1 + w2 + 1e-20
    cmb_ref[...] = (jnp.where(f1, w1, 0.0) + jnp.where(f2, w2, 0.0)) / denom


def _moe_body(x_ref, wgu_ref, wdn_ref, cmb_ref, o_ref):
    e = pl.program_id(1)

    @pl.when(e == 0)
    def _():
        o_ref[...] = jnp.zeros_like(o_ref)

    xb = x_ref[...]
    gu = jnp.dot(xb, wgu_ref[0],
                 preferred_element_type=jnp.float32).astype(jnp.bfloat16)
    g = gu[:, :F]
    h = g * jax.nn.sigmoid(g) * gu[:, F:]
    y = jnp.dot(h, wdn_ref[0], preferred_element_type=jnp.float32)
    onehot = jnp.where(
        jax.lax.broadcasted_iota(jnp.int32, (EHAT, 1), 0) == e, 1.0, 0.0)
    w = jnp.dot(cmb_ref[...], onehot)
    o_ref[...] += y * w


@jax.jit
def kernel(hidden_states, gate_w, e_bias, w_gate_up, w_down, ws_gate_up,
           ws_down):
    x = hidden_states.reshape(T, D)

    combine = pl.pallas_call(
        _router_body,
        out_shape=jax.ShapeDtypeStruct((T, E), jnp.float32),
    )(x, gate_w, e_bias.reshape(1, E))

    sg0 = ws_gate_up[:, 0:F]
    sg1 = ws_gate_up[:, F:FS]
    su0 = ws_gate_up[:, FS:FS + F]
    su1 = ws_gate_up[:, FS + F:]
    shared_gu = jnp.stack(
        [jnp.concatenate([sg0, su0], axis=1),
         jnp.concatenate([sg1, su1], axis=1)])
    wgu = jnp.concatenate([w_gate_up, shared_gu], axis=0).astype(jnp.bfloat16)
    shared_dn = jnp.stack([ws_down[:F], ws_down[F:]])
    wdn = jnp.concatenate([w_down, shared_dn], axis=0).astype(jnp.bfloat16)
    cmb = jnp.concatenate(
        [combine * SCALE, jnp.ones((T, NSHARED), jnp.float32)], axis=1)
    xb = x.astype(jnp.bfloat16)

    out = pl.pallas_call(
        _moe_body,
        grid=(T // BT, EHAT),
        in_specs=[
            pl.BlockSpec((BT, D), lambda i, e: (i, 0)),
            pl.BlockSpec((1, D, 2 * F), lambda i, e: (e, 0, 0)),
            pl.BlockSpec((1, F, D), lambda i, e: (e, 0, 0)),
            pl.BlockSpec((BT, EHAT), lambda i, e: (i, 0)),
        ],
        out_specs=pl.BlockSpec((BT, D), lambda i, e: (i, 0)),
        out_shape=jax.ShapeDtypeStruct((T, D), jnp.float32),
    )(xb, wgu, wdn, cmb)
    return out


# speedup vs baseline: 2.0843x; 1.1637x over previous
import functools
import jax
import jax.numpy as jnp
from jax.experimental import pallas as pl

T = 2048
D = 1024
E = 8
F = 512
TOPK = 2
NG = 2
NSHARED = 2
SCALE = 2.5
FS = F * NSHARED
EHAT = E + NSHARED

BT = 2048


def _lane_cumsum(a):
    cols = [a[:, 0:1]]
    for j in range(1, a.shape[1]):
        cols.append(cols[-1] + a[:, j:j + 1])
    return jnp.concatenate(cols, axis=1)


def _router_body(x_ref, gw_ref, eb_ref, cmb_ref):
    x = x_ref[...]
    logits = jnp.dot(x, gw_ref[...])
    scores = jax.nn.sigmoid(logits)
    sfc = scores + eb_ref[...]

    def top2sum(grp):
        g1 = jnp.max(grp, axis=-1, keepdims=True)
        eq1 = jnp.where(grp == g1, 1.0, 0.0)
        first1 = (eq1 * _lane_cumsum(eq1)) == 1.0
        g2 = jnp.max(jnp.where(first1, -jnp.inf, grp), axis=-1, keepdims=True)
        return g1 + g2

    gs0 = top2sum(sfc[:, :E // NG])
    gs1 = top2sum(sfc[:, E // NG:])
    gsel_f = jnp.where(gs0 >= gs1, 1.0, 0.0)
    gof = (jax.lax.broadcasted_iota(jnp.int32, (T, E), 1) // (E // NG)
           ).astype(jnp.float32)
    emask_f = gsel_f * (1.0 - gof) + (1.0 - gsel_f) * gof

    masked = jnp.where(emask_f > 0.5, sfc, -1e9)
    m1 = jnp.max(masked, axis=-1, keepdims=True)
    e1 = jnp.where(masked == m1, 1.0, 0.0)
    f1 = (e1 * _lane_cumsum(e1)) == 1.0
    masked2 = jnp.where(f1, -jnp.inf, masked)
    m2 = jnp.max(masked2, axis=-1, keepdims=True)
    e2 = jnp.where(masked2 == m2, 1.0, 0.0)
    f2 = (e2 * _lane_cumsum(e2)) == 1.0

    w1 = jnp.sum(jnp.where(f1, scores, 0.0), axis=-1, keepdims=True)
    w2 = jnp.sum(jnp.where(f2, scores, 0.0), axis=-1, keepdims=True)
    denom = w1 + w2 + 1e-20
    cmb_ref[...] = (jnp.where(f1, w1, 0.0) + jnp.where(f2, w2, 0.0)) * (SCALE / denom)


def _moe_body(x_ref, wgur_ref, wdnr_ref, wsgu_ref, wsdn_ref, cmb_ref, o_ref):
    e = pl.program_id(0)

    @pl.when(e == 0)
    def _():
        o_ref[...] = jnp.zeros_like(o_ref)

    @pl.when(e < E)
    def _routed():
        x = x_ref[...]
        gu = jnp.dot(x, wgur_ref[0], preferred_element_type=jnp.float32)
        g = gu[:, :F]
        h = g * jax.nn.sigmoid(g) * gu[:, F:]
        y = jnp.dot(h, wdnr_ref[0], preferred_element_type=jnp.float32)
        onehot = jnp.where(
            jax.lax.broadcasted_iota(jnp.int32, (E, 1), 0) == e, 1.0, 0.0)
        w = jnp.dot(cmb_ref[...], onehot)
        o_ref[...] += y * w

    @pl.when(e == E)
    def _shared():
        x = x_ref[...]
        for hh in range(NSHARED):
            g = jnp.dot(x, wsgu_ref[:, hh * F:(hh + 1) * F],
                        preferred_element_type=jnp.float32)
            u = jnp.dot(x, wsgu_ref[:, FS + hh * F:FS + (hh + 1) * F],
                        preferred_element_type=jnp.float32)
            hq = g * jax.nn.sigmoid(g) * u
            y = jnp.dot(hq, wsdn_ref[hh * F:(hh + 1) * F, :],
                        preferred_element_type=jnp.float32)
            o_ref[...] += y


@jax.jit
def kernel(hidden_states, gate_w, e_bias, w_gate_up, w_down, ws_gate_up,
           ws_down):
    x = hidden_states.reshape(T, D)

    combine = pl.pallas_call(
        _router_body,
        out_shape=jax.ShapeDtypeStruct((T, E), jnp.float32),
    )(x, gate_w, e_bias.reshape(1, E))

    out = pl.pallas_call(
        _moe_body,
        grid=(E + 1,),
        in_specs=[
            pl.BlockSpec((T, D), lambda e: (0, 0)),
            pl.BlockSpec((1, D, 2 * F), lambda e: (jnp.minimum(e, E - 1), 0, 0)),
            pl.BlockSpec((1, F, D), lambda e: (jnp.minimum(e, E - 1), 0, 0)),
            pl.BlockSpec((D, 2 * FS), lambda e: (0, 0)),
            pl.BlockSpec((FS, D), lambda e: (0, 0)),
            pl.BlockSpec((T, E), lambda e: (0, 0)),
        ],
        out_specs=pl.BlockSpec((T, D), lambda e: (0, 0)),
        out_shape=jax.ShapeDtypeStruct((T, D), jnp.float32),
    )(x, w_gate_up, w_down, ws_gate_up, ws_down, combine)
    return out
